# baseline (speedup 1.0000x reference)
import jax
import jax.numpy as jnp
from jax import lax
from jax.experimental import pallas as pl
from jax.experimental.pallas import tpu as pltpu
from jax.experimental.pallas import tpu_sc as plsc

N = 100000
D = 128
G = 512

NC = 2
NS = 16
L = 16

DH = D // NC
NQ = DH // L
AW = DH + L
SEGS = G // NS
BR = 256
GPB = BR // L
NB_FULL = N // BR
NB_MAIN = 24
NB_EXTRA = NB_FULL - NS * NB_MAIN
TAIL = N - NB_FULL * BR
TAIL_BASE = NB_FULL * BR
NBUF = 4
MR = 128
PDIST = 3


def _body(embed_hbm, batch_hbm, w_hbm, out_hbm,
          dbuf0, dbuf1, dbuf2, dbuf3,
          ibuf0, ibuf1, ibuf2, ibuf3,
          zbuf, acc_loc,
          ramp0, ramp1, ramp2, ramp3,
          abuf, wbuf, obuf, acc_sh,
          lsem0, lsem1, lsem2, lsem3, msem):
  c = lax.axis_index("c")
  s = lax.axis_index("s")
  _Z16 = jnp.zeros((L,), jnp.float32)
  iota = lax.iota(jnp.int32, L)
  _E0 = jnp.where(iota == 0, 1.0, 0.0).astype(jnp.float32)
  _E0x16 = jnp.where(iota == 0, 16.0, 0.0).astype(jnp.float32)

  b0 = jnp.where(s < NB_EXTRA, (NB_MAIN + 1) * s,
                 NB_EXTRA + NB_MAIN * s).astype(jnp.int32)

  def zrow(i, _):
    for j in range(AW // L):
      zbuf[i, pl.ds(j * L, L)] = _Z16
    return 0
  lax.fori_loop(0, SEGS, zrow, 0)

  def zloc(i, _):
    for j in range(AW // L):
      acc_loc[i, pl.ds(j * L, L)] = _Z16
    return 0
  lax.fori_loop(0, G, zloc, 0)
  pltpu.sync_copy(zbuf, acc_sh.at[pl.ds(s * SEGS, SEGS), :])
  RAMPS = (ramp0, ramp1, ramp2, ramp3)
  for q, ramp in enumerate(RAMPS):
    for j in range(MR // L):
      ramp[pl.ds(j * L, L)] = iota + (q * MR + j * L)
  plsc.subcore_barrier()

  cols = pl.ds(c * DH, DH)

  def issue_load(k, dbuf, ibuf, lsem):
    base = (b0 + k) * BR
    pltpu.async_copy(embed_hbm.at[pl.ds(base, BR), cols], dbuf, lsem)
    pltpu.async_copy(batch_hbm.at[pl.ds(base, BR)], ibuf, lsem)

  def wait_load(k, dbuf, ibuf, lsem):
    base = (b0 + k) * BR
    pltpu.make_async_copy(embed_hbm.at[pl.ds(base, BR), cols], dbuf,
                          lsem).wait()
    pltpu.make_async_copy(batch_hbm.at[pl.ds(base, BR)], ibuf, lsem).wait()

  bufs = ((dbuf0, ibuf0, lsem0), (dbuf1, ibuf1, lsem1),
          (dbuf2, ibuf2, lsem2), (dbuf3, ibuf3, lsem3))

  def flush(cur, a0, a1, a2, a3, an):
    row = jnp.maximum(cur, 0)
    plsc.addupdate(acc_loc.at[row, pl.ds(0 * L, L)], a0)
    plsc.addupdate(acc_loc.at[row, pl.ds(1 * L, L)], a1)
    plsc.addupdate(acc_loc.at[row, pl.ds(2 * L, L)], a2)
    plsc.addupdate(acc_loc.at[row, pl.ds(3 * L, L)], a3)
    plsc.addupdate(acc_loc.at[row, pl.ds(4 * L, L)], an)

  def process_group(dbuf, i0, r, carry):
    cur, a0, a1, a2, a3, an = carry
    first = i0[0]
    same_all = (first == i0[L - 1]) & (first == cur)

    def fast(cv):
      cur, a0, a1, a2, a3, an = cv
      vs = [a0, a1, a2, a3]
      for q in range(NQ):
        t = [dbuf[r + j, pl.ds(q * L, L)] for j in range(L)]
        while len(t) > 1:
          t = [t[i] + t[i + 1] for i in range(0, len(t), 2)]
        vs[q] = vs[q] + t[0]
      return (cur, vs[0], vs[1], vs[2], vs[3], an + _E0x16)

    def slow(cv):
      cur, a0, a1, a2, a3, an = cv
      for j in range(L):
        seg = i0[j]
        ch = seg != cur

        @pl.when(ch)
        def _():
          flush(cur, a0, a1, a2, a3, an)

        a0 = jnp.where(ch, _Z16, a0) + dbuf[r + j, pl.ds(0 * L, L)]
        a1 = jnp.where(ch, _Z16, a1) + dbuf[r + j, pl.ds(1 * L, L)]
        a2 = jnp.where(ch, _Z16, a2) + dbuf[r + j, pl.ds(2 * L, L)]
        a3 = jnp.where(ch, _Z16, a3) + dbuf[r + j, pl.ds(3 * L, L)]
        an = jnp.where(ch, _Z16, an) + _E0
        cur = seg
      return (cur, a0, a1, a2, a3, an)

    return lax.cond(same_all, fast, slow, carry)

  def process_block(dbuf, ibuf, carry):
    def gbody(gr, cv):
      r = gr * L
      i0 = ibuf[pl.ds(r, L)]
      return process_group(dbuf, i0, r, cv)
    return lax.fori_loop(0, GPB, gbody, carry)

  for kk in range(PDIST):
    issue_load(kk, bufs[kk][0], bufs[kk][1], bufs[kk][2])

  carry0 = (jnp.int32(-1), _Z16, _Z16, _Z16, _Z16, _Z16)

  def step(j, carry):
    for par in range(NBUF):
      dbuf, ibuf, lsem = bufs[par]
      m = NBUF * j + par

      pbuf, pibuf, plsem = bufs[(par + PDIST) % NBUF]

      @pl.when(m + PDIST < NB_MAIN)
      def _():
        issue_load(m + PDIST, pbuf, pibuf, plsem)

      wait_load(m, dbuf, ibuf, lsem)
      carry = process_block(dbuf, ibuf, carry)
    return carry
  carry = lax.fori_loop(0, NB_MAIN // NBUF, step, carry0)

  def do_extra(cv):
    base = (b0 + NB_MAIN) * BR
    pltpu.sync_copy(embed_hbm.at[pl.ds(base, BR), cols], dbuf0)
    pltpu.sync_copy(batch_hbm.at[pl.ds(base, BR)], ibuf0)
    return process_block(dbuf0, ibuf0, cv)
  carry = lax.cond(s < NB_EXTRA, do_extra, lambda cv: cv, carry)

  def do_tail(cv):
    pltpu.sync_copy(embed_hbm.at[pl.ds(TAIL_BASE, TAIL), cols],
                    dbuf0.at[pl.ds(0, TAIL), :])
    pltpu.sync_copy(batch_hbm.at[pl.ds(TAIL_BASE, TAIL)],
                    ibuf0.at[pl.ds(0, TAIL)])

    def tbody(gr, cv2):
      r = gr * L
      i0 = ibuf0[pl.ds(r, L)]
      return process_group(dbuf0, i0, r, cv2)
    return lax.fori_loop(0, TAIL // L, tbody, cv)
  carry = lax.cond(s == NS - 1, do_tail, lambda cv: cv, carry)

  flush(*carry)

  for q, ramp in enumerate(RAMPS):
    pltpu.async_copy(acc_loc.at[pl.ds(q * MR, MR), :], acc_sh.at[ramp],
                     msem, add=True)
  for q, ramp in enumerate(RAMPS):
    pltpu.make_async_copy(acc_loc.at[pl.ds(q * MR, MR), :], acc_sh.at[ramp],
                          msem).wait()

  plsc.subcore_barrier()

  pltpu.sync_copy(w_hbm.at[pl.ds(c * DH, DH)], wbuf)
  pltpu.sync_copy(acc_sh.at[pl.ds(s * SEGS, SEGS), :], abuf)

  for grp in range(SEGS // L):
    rows = iota + grp * L

    def dstep(d, acc_v):
      col = jnp.full((L,), d, jnp.int32)
      return acc_v + (plsc.load_gather(abuf, [rows, col])
                      * plsc.load_gather(wbuf, [col]))
    acc_v = lax.fori_loop(0, DH, dstep, _Z16)
    cnt_v = jnp.maximum(
        plsc.load_gather(abuf, [rows, jnp.full((L,), DH, jnp.int32)]), 1.0)
    obuf[pl.ds(grp * L, L)] = acc_v / cnt_v

  pltpu.sync_copy(obuf, out_hbm.at[c, pl.ds(s * SEGS, SEGS)])


@jax.jit
def _pooled_linear(embed, batch_i32, w_flat):
  mesh = plsc.VectorSubcoreMesh(core_axis_name="c", subcore_axis_name="s",
                                num_cores=NC, num_subcores=NS)
  fn = pl.kernel(
      _body,
      out_type=jax.ShapeDtypeStruct((NC, G), jnp.float32),
      mesh=mesh,
      scratch_types=(
          [pltpu.VMEM((BR, DH), jnp.float32)] * NBUF +
          [pltpu.VMEM((BR,), jnp.int32)] * NBUF +
          [pltpu.VMEM((SEGS, AW), jnp.float32),
           pltpu.VMEM((G, AW), jnp.float32),
           pltpu.VMEM((MR,), jnp.int32),
           pltpu.VMEM((MR,), jnp.int32),
           pltpu.VMEM((MR,), jnp.int32),
           pltpu.VMEM((MR,), jnp.int32),
           pltpu.VMEM((SEGS, AW), jnp.float32),
           pltpu.VMEM((DH,), jnp.float32),
           pltpu.VMEM((SEGS,), jnp.float32),
           pltpu.VMEM_SHARED((G, AW), jnp.float32)] +
          [pltpu.SemaphoreType.DMA] * (NBUF + 1)
      ),
      compiler_params=pltpu.CompilerParams(use_tc_tiling_on_sc=False,
                                           needs_layout_passes=False),
  )
  return fn(embed, batch_i32, w_flat)


def kernel(embed, batch, W, b):
  partials = _pooled_linear(embed, batch.astype(jnp.int32), W.reshape(D))
  return (partials[0] + partials[1] + b[0]).reshape(G, 1)


# speedup vs baseline: 1.0247x; 1.0247x over previous
import jax
import jax.numpy as jnp
from jax import lax
from jax.experimental import pallas as pl
from jax.experimental.pallas import tpu as pltpu
from jax.experimental.pallas import tpu_sc as plsc

N = 100000
D = 128
G = 512

NC = 2
NS = 16
L = 16

DH = D // NC
NQ = DH // L
AW = DH + L
SEGS = G // NS
BR = 256
GPB = BR // L
NB_FULL = N // BR
NB_MAIN = 24
NB_EXTRA = NB_FULL - NS * NB_MAIN
TAIL = N - NB_FULL * BR
TAIL_BASE = NB_FULL * BR
NBUF = 4
MR = 128
PDIST = 2


def _body(embed_hbm, batch_hbm, w_hbm, out_hbm,
          dbuf0, dbuf1, dbuf2, dbuf3,
          ibuf0, ibuf1, ibuf2, ibuf3,
          zbuf, acc_loc,
          ramp0, ramp1, ramp2, ramp3,
          abuf, wbuf, obuf, acc_sh,
          lsem0, lsem1, lsem2, lsem3, msem):
  c = lax.axis_index("c")
  s = lax.axis_index("s")
  _Z16 = jnp.zeros((L,), jnp.float32)
  iota = lax.iota(jnp.int32, L)
  _E0 = jnp.where(iota == 0, 1.0, 0.0).astype(jnp.float32)
  _E0x16 = jnp.where(iota == 0, 16.0, 0.0).astype(jnp.float32)

  b0 = jnp.where(s < NB_EXTRA, (NB_MAIN + 1) * s,
                 NB_EXTRA + NB_MAIN * s).astype(jnp.int32)

  def zrow(i, _):
    for j in range(AW // L):
      zbuf[i, pl.ds(j * L, L)] = _Z16
    return 0
  lax.fori_loop(0, SEGS, zrow, 0)

  def zloc(i, _):
    for j in range(AW // L):
      acc_loc[i, pl.ds(j * L, L)] = _Z16
    return 0
  lax.fori_loop(0, G, zloc, 0)
  pltpu.sync_copy(zbuf, acc_sh.at[pl.ds(s * SEGS, SEGS), :])
  RAMPS = (ramp0, ramp1, ramp2, ramp3)
  for q, ramp in enumerate(RAMPS):
    for j in range(MR // L):
      ramp[pl.ds(j * L, L)] = iota + (q * MR + j * L)
  plsc.subcore_barrier()

  cols = pl.ds(c * DH, DH)

  def issue_load(k, dbuf, ibuf, lsem):
    base = (b0 + k) * BR
    pltpu.async_copy(embed_hbm.at[pl.ds(base, BR), cols], dbuf, lsem)
    pltpu.async_copy(batch_hbm.at[pl.ds(base, BR)], ibuf, lsem)

  def wait_load(k, dbuf, ibuf, lsem):
    base = (b0 + k) * BR
    pltpu.make_async_copy(embed_hbm.at[pl.ds(base, BR), cols], dbuf,
                          lsem).wait()
    pltpu.make_async_copy(batch_hbm.at[pl.ds(base, BR)], ibuf, lsem).wait()

  bufs = ((dbuf0, ibuf0, lsem0), (dbuf1, ibuf1, lsem1),
          (dbuf2, ibuf2, lsem2), (dbuf3, ibuf3, lsem3))

  def flush(cur, a0, a1, a2, a3, an):
    row = jnp.maximum(cur, 0)
    plsc.addupdate(acc_loc.at[row, pl.ds(0 * L, L)], a0)
    plsc.addupdate(acc_loc.at[row, pl.ds(1 * L, L)], a1)
    plsc.addupdate(acc_loc.at[row, pl.ds(2 * L, L)], a2)
    plsc.addupdate(acc_loc.at[row, pl.ds(3 * L, L)], a3)
    plsc.addupdate(acc_loc.at[row, pl.ds(4 * L, L)], an)

  def process_group(dbuf, i0, r, carry):
    cur, a0, a1, a2, a3, an = carry
    first = i0[0]
    same_all = (first == i0[L - 1]) & (first == cur)

    def fast(cv):
      cur, a0, a1, a2, a3, an = cv
      vs = [a0, a1, a2, a3]
      for q in range(NQ):
        t = [dbuf[r + j, pl.ds(q * L, L)] for j in range(L)]
        while len(t) > 1:
          t = [t[i] + t[i + 1] for i in range(0, len(t), 2)]
        vs[q] = vs[q] + t[0]
      return (cur, vs[0], vs[1], vs[2], vs[3], an + _E0x16)

    def slow(cv):
      cur, a0, a1, a2, a3, an = cv
      for j in range(L):
        seg = i0[j]
        ch = seg != cur

        @pl.when(ch)
        def _():
          flush(cur, a0, a1, a2, a3, an)

        a0 = jnp.where(ch, _Z16, a0) + dbuf[r + j, pl.ds(0 * L, L)]
        a1 = jnp.where(ch, _Z16, a1) + dbuf[r + j, pl.ds(1 * L, L)]
        a2 = jnp.where(ch, _Z16, a2) + dbuf[r + j, pl.ds(2 * L, L)]
        a3 = jnp.where(ch, _Z16, a3) + dbuf[r + j, pl.ds(3 * L, L)]
        an = jnp.where(ch, _Z16, an) + _E0
        cur = seg
      return (cur, a0, a1, a2, a3, an)

    return lax.cond(same_all, fast, slow, carry)

  def process_block(dbuf, ibuf, carry):
    def gbody(gr, cv):
      r = gr * L
      i0 = ibuf[pl.ds(r, L)]
      return process_group(dbuf, i0, r, cv)
    return lax.fori_loop(0, GPB, gbody, carry)

  for kk in range(PDIST):
    issue_load(kk, bufs[kk][0], bufs[kk][1], bufs[kk][2])

  carry0 = (jnp.int32(-1), _Z16, _Z16, _Z16, _Z16, _Z16)

  def step(j, carry):
    for par in range(NBUF):
      dbuf, ibuf, lsem = bufs[par]
      m = NBUF * j + par

      wait_load(m, dbuf, ibuf, lsem)

      pbuf, pibuf, plsem = bufs[(par + PDIST) % NBUF]

      @pl.when(m + PDIST < NB_MAIN)
      def _():
        issue_load(m + PDIST, pbuf, pibuf, plsem)

      carry = process_block(dbuf, ibuf, carry)
    return carry
  carry = lax.fori_loop(0, NB_MAIN // NBUF, step, carry0)

  def do_extra(cv):
    base = (b0 + NB_MAIN) * BR
    pltpu.sync_copy(embed_hbm.at[pl.ds(base, BR), cols], dbuf0)
    pltpu.sync_copy(batch_hbm.at[pl.ds(base, BR)], ibuf0)
    return process_block(dbuf0, ibuf0, cv)
  carry = lax.cond(s < NB_EXTRA, do_extra, lambda cv: cv, carry)

  def do_tail(cv):
    pltpu.sync_copy(embed_hbm.at[pl.ds(TAIL_BASE, TAIL), cols],
                    dbuf0.at[pl.ds(0, TAIL), :])
    pltpu.sync_copy(batch_hbm.at[pl.ds(TAIL_BASE, TAIL)],
                    ibuf0.at[pl.ds(0, TAIL)])

    def tbody(gr, cv2):
      r = gr * L
      i0 = ibuf0[pl.ds(r, L)]
      return process_group(dbuf0, i0, r, cv2)
    return lax.fori_loop(0, TAIL // L, tbody, cv)
  carry = lax.cond(s == NS - 1, do_tail, lambda cv: cv, carry)

  flush(*carry)

  for q, ramp in enumerate(RAMPS):
    pltpu.async_copy(acc_loc.at[pl.ds(q * MR, MR), :], acc_sh.at[ramp],
                     msem, add=True)
  for q, ramp in enumerate(RAMPS):
    pltpu.make_async_copy(acc_loc.at[pl.ds(q * MR, MR), :], acc_sh.at[ramp],
                          msem).wait()

  plsc.subcore_barrier()

  pltpu.sync_copy(w_hbm.at[pl.ds(c * DH, DH)], wbuf)
  pltpu.sync_copy(acc_sh.at[pl.ds(s * SEGS, SEGS), :], abuf)

  for grp in range(SEGS // L):
    rows = iota + grp * L

    def dstep(d, acc_v):
      col = jnp.full((L,), d, jnp.int32)
      return acc_v + (plsc.load_gather(abuf, [rows, col])
                      * plsc.load_gather(wbuf, [col]))
    acc_v = lax.fori_loop(0, DH, dstep, _Z16)
    cnt_v = jnp.maximum(
        plsc.load_gather(abuf, [rows, jnp.full((L,), DH, jnp.int32)]), 1.0)
    obuf[pl.ds(grp * L, L)] = acc_v / cnt_v

  pltpu.sync_copy(obuf, out_hbm.at[c, pl.ds(s * SEGS, SEGS)])


@jax.jit
def _pooled_linear(embed, batch_i32, w_flat):
  mesh = plsc.VectorSubcoreMesh(core_axis_name="c", subcore_axis_name="s",
                                num_cores=NC, num_subcores=NS)
  fn = pl.kernel(
      _body,
      out_type=jax.ShapeDtypeStruct((NC, G), jnp.float32),
      mesh=mesh,
      scratch_types=(
          [pltpu.VMEM((BR, DH), jnp.float32)] * NBUF +
          [pltpu.VMEM((BR,), jnp.int32)] * NBUF +
          [pltpu.VMEM((SEGS, AW), jnp.float32),
           pltpu.VMEM((G, AW), jnp.float32),
           pltpu.VMEM((MR,), jnp.int32),
           pltpu.VMEM((MR,), jnp.int32),
           pltpu.VMEM((MR,), jnp.int32),
           pltpu.VMEM((MR,), jnp.int32),
           pltpu.VMEM((SEGS, AW), jnp.float32),
           pltpu.VMEM((DH,), jnp.float32),
           pltpu.VMEM((SEGS,), jnp.float32),
           pltpu.VMEM_SHARED((G, AW), jnp.float32)] +
          [pltpu.SemaphoreType.DMA] * (NBUF + 1)
      ),
      compiler_params=pltpu.CompilerParams(use_tc_tiling_on_sc=False,
                                           needs_layout_passes=False),
  )
  return fn(embed, batch_i32, w_flat)


def kernel(embed, batch, W, b):
  partials = _pooled_linear(embed, batch.astype(jnp.int32), W.reshape(D))
  return (partials[0] + partials[1] + b[0]).reshape(G, 1)
